# baseline (speedup 1.0000x reference)
import functools

import jax
import jax.numpy as jnp
from jax import lax
from jax.experimental import pallas as pl
from jax.experimental.pallas import tpu as pltpu
from jax.experimental.pallas import tpu_sc as plsc

EPS_ = 1e-12
NT_ = 1000
NSPLIT_ = 8
NSKILL_ = 64
D_ = NSPLIT_ * NSKILL_
B_ = 16384
NLANE_ = 16
NVR_ = NSKILL_ // NLANE_

NC_ = 2
NS_ = 16
NW_ = NC_ * NS_
BPW_ = B_ // NW_
CH_ = 32
NCH_ = BPW_ // CH_
NBUF_ = 2


def _norm_body(x_ref, o_ref):
    x = x_ref[:]
    p = 1.0 / (1.0 + jnp.exp(-x))
    j = lax.broadcasted_iota(jnp.int32, (D_, NSPLIT_), 0) // NSKILL_
    k = lax.broadcasted_iota(jnp.int32, (D_, NSPLIT_), 1)
    s_mat = (j == k).astype(jnp.float32)
    denom = jnp.dot(p, s_mat, preferred_element_type=jnp.float32)
    jj = lax.broadcasted_iota(jnp.int32, (NSPLIT_, D_), 1) // NSKILL_
    kk = lax.broadcasted_iota(jnp.int32, (NSPLIT_, D_), 0)
    e_mat = (jj == kk).astype(jnp.float32)
    dnb = jnp.dot(denom, e_mat, preferred_element_type=jnp.float32)
    o_ref[:] = p / (dnb + EPS_)


def _normalize_table(module_logits):
    return pl.pallas_call(
        _norm_body,
        out_shape=jax.ShapeDtypeStruct((NT_, D_), jnp.float32),
    )(module_logits)


def _gather_body(table_hbm, idx_hbm, out_hbm, idx_v,
                 r0, r1, t0, t1, g0, g1, o0, o1):
    bufs2 = (r0, r1)
    bufs3 = (t0, t1)
    gsem = (g0, g1)
    osem = (o0, o1)
    wid = lax.axis_index("s") * NC_ + lax.axis_index("c")
    base = wid * BPW_
    pltpu.sync_copy(idx_hbm.at[pl.ds(base, BPW_)], idx_v)

    def convert(b):
        src, dst = bufs2[b], bufs3[b]

        def row(r, carry):
            for s in range(NSPLIT_):
                for v in range(NVR_):
                    dst[r, s, pl.ds(v * NLANE_, NLANE_)] = (
                        src[r, pl.ds(s * NSKILL_ + v * NLANE_, NLANE_)])
            return carry

        lax.fori_loop(0, CH_, row, 0)

    gh = [None] * NCH_
    oh = [None] * NCH_
    for c in range(NBUF_):
        gh[c] = pltpu.async_copy(
            table_hbm.at[idx_v.at[pl.ds(c * CH_, CH_)]], bufs2[c], gsem[c]
        )
    for c in range(NCH_):
        b = c % NBUF_
        gh[c].wait()
        if c >= NBUF_:
            oh[c - NBUF_].wait()
        convert(b)
        oh[c] = pltpu.async_copy(
            bufs3[b], out_hbm.at[pl.ds(base + c * CH_, CH_)], osem[b]
        )
        nxt = c + NBUF_
        if nxt < NCH_:
            gh[nxt] = pltpu.async_copy(
                table_hbm.at[idx_v.at[pl.ds(nxt * CH_, CH_)]], bufs2[b], gsem[b]
            )
    for c in range(max(0, NCH_ - NBUF_), NCH_):
        oh[c].wait()


_sc_gather = functools.partial(
    pl.kernel,
    mesh=plsc.VectorSubcoreMesh(core_axis_name="c", subcore_axis_name="s"),
    out_type=jax.ShapeDtypeStruct((B_, NSPLIT_, NSKILL_), jnp.float32),
    scratch_types=[
        pltpu.VMEM((BPW_,), jnp.int32),
        pltpu.VMEM((CH_, D_), jnp.float32),
        pltpu.VMEM((CH_, D_), jnp.float32),
        pltpu.VMEM((CH_, NSPLIT_, NSKILL_), jnp.float32),
        pltpu.VMEM((CH_, NSPLIT_, NSKILL_), jnp.float32),
        pltpu.SemaphoreType.DMA,
        pltpu.SemaphoreType.DMA,
        pltpu.SemaphoreType.DMA,
        pltpu.SemaphoreType.DMA,
    ],
)(_gather_body)


def kernel(task_ids, input_ids, module_logits):
    del input_ids
    table = _normalize_table(module_logits)
    return _sc_gather(table, task_ids.astype(jnp.int32))


# speedup vs baseline: 1.1711x; 1.1711x over previous
import functools

import jax
import jax.numpy as jnp
from jax import lax
from jax.experimental import pallas as pl
from jax.experimental.pallas import tpu as pltpu
from jax.experimental.pallas import tpu_sc as plsc

EPS_ = 1e-12
NT_ = 1000
NSPLIT_ = 8
NSKILL_ = 64
D_ = NSPLIT_ * NSKILL_
B_ = 16384
NLANE_ = 16
NVR_ = NSKILL_ // NLANE_

NC_ = 2
NS_ = 16
NW_ = NC_ * NS_
BPW_ = B_ // NW_
CH_ = 32
NCH_ = BPW_ // CH_
NBUF_ = 2


def _norm_body(x_ref, o_ref):
    x = x_ref[:]
    p = 1.0 / (1.0 + jnp.exp(-x))
    j = lax.broadcasted_iota(jnp.int32, (D_, NSPLIT_), 0) // NSKILL_
    k = lax.broadcasted_iota(jnp.int32, (D_, NSPLIT_), 1)
    s_mat = (j == k).astype(jnp.float32)
    denom = jnp.dot(p, s_mat, preferred_element_type=jnp.float32)
    jj = lax.broadcasted_iota(jnp.int32, (NSPLIT_, D_), 1) // NSKILL_
    kk = lax.broadcasted_iota(jnp.int32, (NSPLIT_, D_), 0)
    e_mat = (jj == kk).astype(jnp.float32)
    dnb = jnp.dot(denom, e_mat, preferred_element_type=jnp.float32)
    o_ref[:] = p / (dnb + EPS_)


def _normalize_table(module_logits):
    return pl.pallas_call(
        _norm_body,
        out_shape=jax.ShapeDtypeStruct((NT_, D_), jnp.float32),
    )(module_logits)


def _gather_body(table_hbm, idx_hbm, out_hbm, idx_v,
                 r0, r1, t0, t1, g0, g1, o0, o1):
    bufs2 = (r0, r1)
    bufs3 = (t0, t1)
    gsem = (g0, g1)
    osem = (o0, o1)
    wid = lax.axis_index("s") * NC_ + lax.axis_index("c")
    base = wid * BPW_
    pltpu.sync_copy(idx_hbm.at[pl.ds(base, BPW_)], idx_v)

    def convert(b):
        src, dst = bufs2[b], bufs3[b]

        @plsc.parallel_loop(0, CH_, 1, unroll=2)
        def row(r):
            for s in range(NSPLIT_):
                for v in range(NVR_):
                    dst[r, s, pl.ds(v * NLANE_, NLANE_)] = (
                        src[r, pl.ds(s * NSKILL_ + v * NLANE_, NLANE_)])

    gh = [None] * NCH_
    oh = [None] * NCH_
    for c in range(NBUF_):
        gh[c] = pltpu.async_copy(
            table_hbm.at[idx_v.at[pl.ds(c * CH_, CH_)]], bufs2[c], gsem[c]
        )
    for c in range(NCH_):
        b = c % NBUF_
        gh[c].wait()
        if c >= NBUF_:
            oh[c - NBUF_].wait()
        convert(b)
        oh[c] = pltpu.async_copy(
            bufs3[b], out_hbm.at[pl.ds(base + c * CH_, CH_)], osem[b]
        )
        nxt = c + NBUF_
        if nxt < NCH_:
            gh[nxt] = pltpu.async_copy(
                table_hbm.at[idx_v.at[pl.ds(nxt * CH_, CH_)]], bufs2[b], gsem[b]
            )
    for c in range(max(0, NCH_ - NBUF_), NCH_):
        oh[c].wait()


_sc_gather = functools.partial(
    pl.kernel,
    mesh=plsc.VectorSubcoreMesh(core_axis_name="c", subcore_axis_name="s"),
    out_type=jax.ShapeDtypeStruct((B_, NSPLIT_, NSKILL_), jnp.float32),
    scratch_types=[
        pltpu.VMEM((BPW_,), jnp.int32),
        pltpu.VMEM((CH_, D_), jnp.float32),
        pltpu.VMEM((CH_, D_), jnp.float32),
        pltpu.VMEM((CH_, NSPLIT_, NSKILL_), jnp.float32),
        pltpu.VMEM((CH_, NSPLIT_, NSKILL_), jnp.float32),
        pltpu.SemaphoreType.DMA,
        pltpu.SemaphoreType.DMA,
        pltpu.SemaphoreType.DMA,
        pltpu.SemaphoreType.DMA,
    ],
)(_gather_body)


def kernel(task_ids, input_ids, module_logits):
    del input_ids
    table = _normalize_table(module_logits)
    return _sc_gather(table, task_ids.astype(jnp.int32))
